# Initial kernel scaffold; baseline (speedup 1.0000x reference)
import functools

import jax
import jax.numpy as jnp
from jax import lax
from jax.experimental import pallas as pl
from jax.experimental.pallas import tpu as pltpu
from jax.experimental.pallas import tpu_sc as plsc

N_NODES = 10000
N_PAD = 10240
N_EDGES = 320000
NW = 32
NB = 79
E_PAD = NW * NB * 128
D_FEAT = 128
D_HID = 16
N_CLASSES = 7
ROWS_PER_TILE = N_PAD // 16


def _sc_mesh():
    return plsc.VectorSubcoreMesh(
        core_axis_name="c", subcore_axis_name="s", num_cores=2, num_subcores=16
    )


def _deg_body(dstg, degp, dst_v, ones_v, zbuf_v, deg_sh, sem):
    c = lax.axis_index("c")
    s = lax.axis_index("s")
    wid = c * 16 + s

    def fill_ones(i, carry):
        ones_v[pl.ds(i * 16, 16)] = jnp.ones((16,), jnp.float32)
        return carry

    lax.fori_loop(0, 8, fill_ones, 0)

    def fill_zeros(i, carry):
        zbuf_v[pl.ds(i * 16, 16)] = jnp.zeros((16,), jnp.float32)
        return carry

    lax.fori_loop(0, ROWS_PER_TILE // 16, fill_zeros, 0)

    pltpu.sync_copy(dstg.at[wid], dst_v)
    pltpu.sync_copy(zbuf_v, deg_sh.at[pl.ds(s * ROWS_PER_TILE, ROWS_PER_TILE)])
    plsc.subcore_barrier()

    def body(k, carry):
        pltpu.sync_copy(ones_v, deg_sh.at[dst_v.at[k]], add=True)
        return carry

    lax.fori_loop(0, NB, body, 0)
    plsc.subcore_barrier()

    pltpu.sync_copy(deg_sh.at[pl.ds(s * ROWS_PER_TILE, ROWS_PER_TILE)], zbuf_v)
    pltpu.sync_copy(zbuf_v, degp.at[c, pl.ds(s * ROWS_PER_TILE, ROWS_PER_TILE)])


def _sc_degree(dstg):
    kern = functools.partial(
        pl.kernel,
        out_type=jax.ShapeDtypeStruct((2, N_PAD), jnp.float32),
        mesh=_sc_mesh(),
        scratch_types=[
            pltpu.VMEM((NB, 128), jnp.int32),
            pltpu.VMEM((128,), jnp.float32),
            pltpu.VMEM((ROWS_PER_TILE,), jnp.float32),
            pltpu.VMEM_SHARED((N_PAD,), jnp.float32),
            pltpu.SemaphoreType.DMA,
        ],
    )(_deg_body)
    return kern(dstg)


def _agg_body(table, srcg, dstg, aggp, src_v, dst_v, rows_v, zbuf_v, acc_sh, sem):
    c = lax.axis_index("c")
    s = lax.axis_index("s")
    wid = c * 16 + s

    def fill_zeros(i, carry):
        zbuf_v[i, :] = jnp.zeros((16,), jnp.float32)
        return carry

    lax.fori_loop(0, ROWS_PER_TILE, fill_zeros, 0)

    pltpu.sync_copy(srcg.at[wid], src_v)
    pltpu.sync_copy(dstg.at[wid], dst_v)
    pltpu.sync_copy(zbuf_v, acc_sh.at[pl.ds(s * ROWS_PER_TILE, ROWS_PER_TILE)])
    plsc.subcore_barrier()

    def body(k, carry):
        pltpu.async_copy(table.at[src_v.at[k]], rows_v, sem).wait()
        pltpu.sync_copy(rows_v, acc_sh.at[dst_v.at[k]], add=True)
        return carry

    lax.fori_loop(0, NB, body, 0)
    plsc.subcore_barrier()

    pltpu.sync_copy(acc_sh.at[pl.ds(s * ROWS_PER_TILE, ROWS_PER_TILE)], zbuf_v)
    pltpu.sync_copy(zbuf_v, aggp.at[c, pl.ds(s * ROWS_PER_TILE, ROWS_PER_TILE)])


def _sc_aggregate(table, srcg, dstg):
    kern = functools.partial(
        pl.kernel,
        out_type=jax.ShapeDtypeStruct((2, N_PAD, D_HID), jnp.float32),
        mesh=_sc_mesh(),
        scratch_types=[
            pltpu.VMEM((NB, 128), jnp.int32),
            pltpu.VMEM((NB, 128), jnp.int32),
            pltpu.VMEM((128, D_HID), jnp.float32),
            pltpu.VMEM((ROWS_PER_TILE, D_HID), jnp.float32),
            pltpu.VMEM_SHARED((N_PAD, D_HID), jnp.float32),
            pltpu.SemaphoreType.DMA,
        ],
    )(_agg_body)
    return kern(table, srcg, dstg)


def _tc_dinv_body(degp_ref, dinv_ref):
    deg = degp_ref[0:1, :] + degp_ref[1:2, :] + 1.0
    dinv_ref[...] = lax.rsqrt(deg)


def _tc_table1_body(x_ref, w1_ref, dinv_ref, out_ref):
    h = jnp.dot(x_ref[...], w1_ref[...], preferred_element_type=jnp.float32)
    out_ref[...] = h * dinv_ref[...]


def _tc_table2_body(aggp_ref, t1_ref, dinv_ref, b1_ref, w2_ref, out_ref):
    agg = aggp_ref[0, :, :] + aggp_ref[1, :, :] + t1_ref[...]
    z1 = jnp.maximum(agg * dinv_ref[...] + b1_ref[...], 0.0)
    h2 = jnp.dot(z1, w2_ref[...], preferred_element_type=jnp.float32)
    out_ref[...] = h2 * dinv_ref[...]


def _tc_out_body(aggp_ref, t2_ref, dinv_ref, b2_ref, out_ref):
    agg = aggp_ref[0, :, :] + aggp_ref[1, :, :] + t2_ref[...]
    z = agg * dinv_ref[...] + b2_ref[...]
    col = lax.broadcasted_iota(jnp.int32, (N_PAD, D_HID), 1)
    zm = jnp.where(col < N_CLASSES, z, -1e30)
    m = jnp.max(zm, axis=1, keepdims=True)
    ssum = jnp.sum(jnp.exp(zm - m), axis=1, keepdims=True)
    out_ref[...] = z - m - jnp.log(ssum)


def kernel(x, edge_index, W1, b1, W2, b2):
    src = edge_index[0]
    dst = edge_index[1]
    pad = jnp.full((E_PAD - N_EDGES,), N_PAD - 1, dtype=jnp.int32)
    srcg = jnp.concatenate([src, pad]).reshape(NW, NB, 128)
    dstg = jnp.concatenate([dst, pad]).reshape(NW, NB, 128)
    x_pad = jnp.pad(x, ((0, N_PAD - N_NODES), (0, 0)))
    w2p = jnp.pad(W2, ((0, 0), (0, D_HID - N_CLASSES)))
    b1r = b1.reshape(1, D_HID)
    b2r = jnp.pad(b2, (0, D_HID - N_CLASSES)).reshape(1, D_HID)

    degp = _sc_degree(dstg)

    dinv_row = pl.pallas_call(
        _tc_dinv_body,
        out_shape=jax.ShapeDtypeStruct((1, N_PAD), jnp.float32),
    )(degp)
    dinv = dinv_row.reshape(N_PAD, 1)

    table1 = pl.pallas_call(
        _tc_table1_body,
        out_shape=jax.ShapeDtypeStruct((N_PAD, D_HID), jnp.float32),
    )(x_pad, W1, dinv)

    agg1p = _sc_aggregate(table1, srcg, dstg)

    table2 = pl.pallas_call(
        _tc_table2_body,
        out_shape=jax.ShapeDtypeStruct((N_PAD, D_HID), jnp.float32),
    )(agg1p, table1, dinv, b1r, w2p)

    agg2p = _sc_aggregate(table2, srcg, dstg)

    out16 = pl.pallas_call(
        _tc_out_body,
        out_shape=jax.ShapeDtypeStruct((N_PAD, D_HID), jnp.float32),
    )(agg2p, table2, dinv, b2r)

    return out16[:N_NODES, :N_CLASSES]


# speedup vs baseline: 32.2003x; 32.2003x over previous
import functools

import jax
import jax.numpy as jnp
from jax import lax
from jax.experimental import pallas as pl
from jax.experimental.pallas import tpu as pltpu
from jax.experimental.pallas import tpu_sc as plsc

N_NODES = 10000
N_PAD = 10240
N_EDGES = 320000
NW = 32
NB = 79
E_PAD = NW * NB * 128
D_FEAT = 128
D_HID = 16
N_CLASSES = 7
ROWS_PER_TILE = N_PAD // 16


def _sc_mesh():
    return plsc.VectorSubcoreMesh(
        core_axis_name="c", subcore_axis_name="s", num_cores=2, num_subcores=16
    )


def _deg_body(dstg, degp, dst_v, ones_v, zbuf_v, deg_sh, sem):
    c = lax.axis_index("c")
    s = lax.axis_index("s")
    wid = c * 16 + s

    def fill_ones(i, carry):
        ones_v[pl.ds(i * 16, 16)] = jnp.ones((16,), jnp.float32)
        return carry

    lax.fori_loop(0, 8, fill_ones, 0)

    def fill_zeros(i, carry):
        zbuf_v[pl.ds(i * 16, 16)] = jnp.zeros((16,), jnp.float32)
        return carry

    lax.fori_loop(0, ROWS_PER_TILE // 16, fill_zeros, 0)

    pltpu.sync_copy(dstg.at[wid], dst_v)
    pltpu.sync_copy(zbuf_v, deg_sh.at[pl.ds(s * ROWS_PER_TILE, ROWS_PER_TILE)])
    plsc.subcore_barrier()

    def body(k, carry):
        pltpu.sync_copy(ones_v, deg_sh.at[dst_v.at[k]], add=True)
        return carry

    lax.fori_loop(0, NB, body, 0)
    plsc.subcore_barrier()

    pltpu.sync_copy(deg_sh.at[pl.ds(s * ROWS_PER_TILE, ROWS_PER_TILE)], zbuf_v)
    pltpu.sync_copy(zbuf_v, degp.at[c, pl.ds(s * ROWS_PER_TILE, ROWS_PER_TILE)])


def _sc_degree(dstg):
    kern = functools.partial(
        pl.kernel,
        out_type=jax.ShapeDtypeStruct((2, N_PAD), jnp.float32),
        mesh=_sc_mesh(),
        scratch_types=[
            pltpu.VMEM((NB, 128), jnp.int32),
            pltpu.VMEM((128,), jnp.float32),
            pltpu.VMEM((ROWS_PER_TILE,), jnp.float32),
            pltpu.VMEM_SHARED((N_PAD,), jnp.float32),
            pltpu.SemaphoreType.DMA,
        ],
        compiler_params=pltpu.CompilerParams(use_tc_tiling_on_sc=False),
    )(_deg_body)
    return kern(dstg)


def _agg_body(table, srcg, dstg, aggp, src_v, dst_v, rows_v, zbuf_v, acc_sh, sem):
    c = lax.axis_index("c")
    s = lax.axis_index("s")
    wid = c * 16 + s

    def fill_zeros(i, carry):
        zbuf_v[i, :] = jnp.zeros((16,), jnp.float32)
        return carry

    lax.fori_loop(0, ROWS_PER_TILE, fill_zeros, 0)

    pltpu.sync_copy(srcg.at[wid], src_v)
    pltpu.sync_copy(dstg.at[wid], dst_v)
    pltpu.sync_copy(zbuf_v, acc_sh.at[pl.ds(s * ROWS_PER_TILE, ROWS_PER_TILE)])
    plsc.subcore_barrier()

    def body(k, carry):
        pltpu.async_copy(table.at[src_v.at[k]], rows_v, sem).wait()
        pltpu.sync_copy(rows_v, acc_sh.at[dst_v.at[k]], add=True)
        return carry

    lax.fori_loop(0, NB, body, 0)
    plsc.subcore_barrier()

    pltpu.sync_copy(acc_sh.at[pl.ds(s * ROWS_PER_TILE, ROWS_PER_TILE)], zbuf_v)
    pltpu.sync_copy(zbuf_v, aggp.at[c, pl.ds(s * ROWS_PER_TILE, ROWS_PER_TILE)])


def _sc_aggregate(table, srcg, dstg):
    kern = functools.partial(
        pl.kernel,
        out_type=jax.ShapeDtypeStruct((2, N_PAD, D_HID), jnp.float32),
        mesh=_sc_mesh(),
        scratch_types=[
            pltpu.VMEM((NB, 128), jnp.int32),
            pltpu.VMEM((NB, 128), jnp.int32),
            pltpu.VMEM((128, D_HID), jnp.float32),
            pltpu.VMEM((ROWS_PER_TILE, D_HID), jnp.float32),
            pltpu.VMEM_SHARED((N_PAD, D_HID), jnp.float32),
            pltpu.SemaphoreType.DMA,
        ],
        compiler_params=pltpu.CompilerParams(use_tc_tiling_on_sc=False),
    )(_agg_body)
    return kern(table, srcg, dstg)


def _tc_dinv_body(degp_ref, dinv_ref):
    deg = degp_ref[0:1, :] + degp_ref[1:2, :] + 1.0
    dinv_ref[...] = lax.rsqrt(deg)


def _tc_table1_body(x_ref, w1_ref, dinv_ref, out_ref):
    h = jnp.dot(x_ref[...], w1_ref[...], preferred_element_type=jnp.float32)
    out_ref[...] = h * dinv_ref[...]


def _tc_table2_body(aggp_ref, t1_ref, dinv_ref, b1_ref, w2_ref, out_ref):
    agg = aggp_ref[0, :, :] + aggp_ref[1, :, :] + t1_ref[...]
    z1 = jnp.maximum(agg * dinv_ref[...] + b1_ref[...], 0.0)
    h2 = jnp.dot(z1, w2_ref[...], preferred_element_type=jnp.float32)
    out_ref[...] = h2 * dinv_ref[...]


def _tc_out_body(aggp_ref, t2_ref, dinv_ref, b2_ref, out_ref):
    agg = aggp_ref[0, :, :] + aggp_ref[1, :, :] + t2_ref[...]
    z = agg * dinv_ref[...] + b2_ref[...]
    col = lax.broadcasted_iota(jnp.int32, (N_PAD, D_HID), 1)
    zm = jnp.where(col < N_CLASSES, z, -1e30)
    m = jnp.max(zm, axis=1, keepdims=True)
    ssum = jnp.sum(jnp.exp(zm - m), axis=1, keepdims=True)
    out_ref[...] = z - m - jnp.log(ssum)


def kernel(x, edge_index, W1, b1, W2, b2):
    src = edge_index[0]
    dst = edge_index[1]
    pad = jnp.full((E_PAD - N_EDGES,), N_PAD - 1, dtype=jnp.int32)
    srcg = jnp.concatenate([src, pad]).reshape(NW, NB, 128)
    dstg = jnp.concatenate([dst, pad]).reshape(NW, NB, 128)
    x_pad = jnp.pad(x, ((0, N_PAD - N_NODES), (0, 0)))
    w2p = jnp.pad(W2, ((0, 0), (0, D_HID - N_CLASSES)))
    b1r = b1.reshape(1, D_HID)
    b2r = jnp.pad(b2, (0, D_HID - N_CLASSES)).reshape(1, D_HID)

    degp = _sc_degree(dstg)

    dinv_row = pl.pallas_call(
        _tc_dinv_body,
        out_shape=jax.ShapeDtypeStruct((1, N_PAD), jnp.float32),
    )(degp)
    dinv = dinv_row.reshape(N_PAD, 1)

    table1 = pl.pallas_call(
        _tc_table1_body,
        out_shape=jax.ShapeDtypeStruct((N_PAD, D_HID), jnp.float32),
    )(x_pad, W1, dinv)

    agg1p = _sc_aggregate(table1, srcg, dstg)

    table2 = pl.pallas_call(
        _tc_table2_body,
        out_shape=jax.ShapeDtypeStruct((N_PAD, D_HID), jnp.float32),
    )(agg1p, table1, dinv, b1r, w2p)

    agg2p = _sc_aggregate(table2, srcg, dstg)

    out16 = pl.pallas_call(
        _tc_out_body,
        out_shape=jax.ShapeDtypeStruct((N_PAD, D_HID), jnp.float32),
    )(agg2p, table2, dinv, b2r)

    return out16[:N_NODES, :N_CLASSES]


# speedup vs baseline: 38.5744x; 1.1980x over previous
import functools

import jax
import jax.numpy as jnp
from jax import lax
from jax.experimental import pallas as pl
from jax.experimental.pallas import tpu as pltpu
from jax.experimental.pallas import tpu_sc as plsc

N_NODES = 10000
N_PAD = 10240
N_EDGES = 320000
NW = 32
K = 8
G = 10
NB = K * G
E_PAD = NW * NB * 128
D_FEAT = 128
D_HID = 16
N_CLASSES = 7
ROWS_PER_TILE = N_PAD // 16


def _sc_mesh():
    return plsc.VectorSubcoreMesh(
        core_axis_name="c", subcore_axis_name="s", num_cores=2, num_subcores=16
    )


def _deg_body(dstg, degp, dst_v, ones_v, zbuf_v, deg_sh, sem):
    c = lax.axis_index("c")
    s = lax.axis_index("s")
    wid = c * 16 + s

    def fill_ones(i, carry):
        ones_v[pl.ds(i * 16, 16)] = jnp.ones((16,), jnp.float32)
        return carry

    lax.fori_loop(0, 8, fill_ones, 0)

    def fill_zeros(i, carry):
        zbuf_v[pl.ds(i * 16, 16)] = jnp.zeros((16,), jnp.float32)
        return carry

    lax.fori_loop(0, ROWS_PER_TILE // 16, fill_zeros, 0)

    pltpu.sync_copy(dstg.at[wid], dst_v)
    pltpu.sync_copy(zbuf_v, deg_sh.at[pl.ds(s * ROWS_PER_TILE, ROWS_PER_TILE)])
    plsc.subcore_barrier()

    def body(g, carry):
        descs = [
            pltpu.async_copy(ones_v, deg_sh.at[dst_v.at[g * K + b]], sem, add=True)
            for b in range(K)
        ]
        for d in descs:
            d.wait()
        return carry

    lax.fori_loop(0, G, body, 0)
    plsc.subcore_barrier()

    pltpu.sync_copy(deg_sh.at[pl.ds(s * ROWS_PER_TILE, ROWS_PER_TILE)], zbuf_v)
    pltpu.sync_copy(zbuf_v, degp.at[c, pl.ds(s * ROWS_PER_TILE, ROWS_PER_TILE)])


def _sc_degree(dstg):
    kern = functools.partial(
        pl.kernel,
        out_type=jax.ShapeDtypeStruct((2, N_PAD), jnp.float32),
        mesh=_sc_mesh(),
        scratch_types=[
            pltpu.VMEM((NB, 128), jnp.int32),
            pltpu.VMEM((128,), jnp.float32),
            pltpu.VMEM((ROWS_PER_TILE,), jnp.float32),
            pltpu.VMEM_SHARED((N_PAD,), jnp.float32),
            pltpu.SemaphoreType.DMA,
        ],
        compiler_params=pltpu.CompilerParams(use_tc_tiling_on_sc=False),
    )(_deg_body)
    return kern(dstg)


def _agg_body(table, srcg, dstg, aggp, src_v, dst_v, rows_v, zbuf_v, acc_sh,
              gsem, ssem):
    c = lax.axis_index("c")
    s = lax.axis_index("s")
    wid = c * 16 + s

    def fill_zeros(i, carry):
        zbuf_v[i, :] = jnp.zeros((16,), jnp.float32)
        return carry

    lax.fori_loop(0, ROWS_PER_TILE, fill_zeros, 0)

    pltpu.sync_copy(srcg.at[wid], src_v)
    pltpu.sync_copy(dstg.at[wid], dst_v)
    pltpu.sync_copy(zbuf_v, acc_sh.at[pl.ds(s * ROWS_PER_TILE, ROWS_PER_TILE)])
    plsc.subcore_barrier()

    def issue_gathers(g, setidx):
        for b in range(K):
            pltpu.async_copy(
                table.at[src_v.at[g * K + b]], rows_v.at[setidx, b], gsem
            )

    issue_gathers(0, 0)

    def body(g, carry):
        cur = lax.rem(g, 2)
        nxt = lax.rem(g + 1, 2)

        @pl.when(g + 1 < G)
        def _():
            issue_gathers(g + 1, nxt)

        for b in range(K):
            pltpu.make_async_copy(
                table.at[src_v.at[g * K + b]], rows_v.at[cur, b], gsem
            ).wait()
        descs = [
            pltpu.async_copy(
                rows_v.at[cur, b], acc_sh.at[dst_v.at[g * K + b]], ssem, add=True
            )
            for b in range(K)
        ]
        for d in descs:
            d.wait()
        return carry

    lax.fori_loop(0, G, body, 0)
    plsc.subcore_barrier()

    pltpu.sync_copy(acc_sh.at[pl.ds(s * ROWS_PER_TILE, ROWS_PER_TILE)], zbuf_v)
    pltpu.sync_copy(zbuf_v, aggp.at[c, pl.ds(s * ROWS_PER_TILE, ROWS_PER_TILE)])


def _sc_aggregate(table, srcg, dstg):
    kern = functools.partial(
        pl.kernel,
        out_type=jax.ShapeDtypeStruct((2, N_PAD, D_HID), jnp.float32),
        mesh=_sc_mesh(),
        scratch_types=[
            pltpu.VMEM((NB, 128), jnp.int32),
            pltpu.VMEM((NB, 128), jnp.int32),
            pltpu.VMEM((2, K, 128, D_HID), jnp.float32),
            pltpu.VMEM((ROWS_PER_TILE, D_HID), jnp.float32),
            pltpu.VMEM_SHARED((N_PAD, D_HID), jnp.float32),
            pltpu.SemaphoreType.DMA,
            pltpu.SemaphoreType.DMA,
        ],
        compiler_params=pltpu.CompilerParams(use_tc_tiling_on_sc=False),
    )(_agg_body)
    return kern(table, srcg, dstg)


def _tc_dinv_body(degp_ref, dinv_ref):
    deg = degp_ref[0:1, :] + degp_ref[1:2, :] + 1.0
    dinv_ref[...] = lax.rsqrt(deg)


def _tc_table1_body(x_ref, w1_ref, dinv_ref, out_ref):
    h = jnp.dot(x_ref[...], w1_ref[...], preferred_element_type=jnp.float32)
    out_ref[...] = h * dinv_ref[...]


def _tc_table2_body(aggp_ref, t1_ref, dinv_ref, b1_ref, w2_ref, out_ref):
    agg = aggp_ref[0, :, :] + aggp_ref[1, :, :] + t1_ref[...]
    z1 = jnp.maximum(agg * dinv_ref[...] + b1_ref[...], 0.0)
    h2 = jnp.dot(z1, w2_ref[...], preferred_element_type=jnp.float32)
    out_ref[...] = h2 * dinv_ref[...]


def _tc_out_body(aggp_ref, t2_ref, dinv_ref, b2_ref, out_ref):
    agg = aggp_ref[0, :, :] + aggp_ref[1, :, :] + t2_ref[...]
    z = agg * dinv_ref[...] + b2_ref[...]
    col = lax.broadcasted_iota(jnp.int32, (N_PAD, D_HID), 1)
    zm = jnp.where(col < N_CLASSES, z, -1e30)
    m = jnp.max(zm, axis=1, keepdims=True)
    ssum = jnp.sum(jnp.exp(zm - m), axis=1, keepdims=True)
    out_ref[...] = z - m - jnp.log(ssum)


def kernel(x, edge_index, W1, b1, W2, b2):
    src = edge_index[0]
    dst = edge_index[1]
    pad = jnp.full((E_PAD - N_EDGES,), N_PAD - 1, dtype=jnp.int32)
    srcg = jnp.concatenate([src, pad]).reshape(NW, NB, 128)
    dstg = jnp.concatenate([dst, pad]).reshape(NW, NB, 128)
    x_pad = jnp.pad(x, ((0, N_PAD - N_NODES), (0, 0)))
    w2p = jnp.pad(W2, ((0, 0), (0, D_HID - N_CLASSES)))
    b1r = b1.reshape(1, D_HID)
    b2r = jnp.pad(b2, (0, D_HID - N_CLASSES)).reshape(1, D_HID)

    degp = _sc_degree(dstg)

    dinv_row = pl.pallas_call(
        _tc_dinv_body,
        out_shape=jax.ShapeDtypeStruct((1, N_PAD), jnp.float32),
    )(degp)
    dinv = dinv_row.reshape(N_PAD, 1)

    table1 = pl.pallas_call(
        _tc_table1_body,
        out_shape=jax.ShapeDtypeStruct((N_PAD, D_HID), jnp.float32),
    )(x_pad, W1, dinv)

    agg1p = _sc_aggregate(table1, srcg, dstg)

    table2 = pl.pallas_call(
        _tc_table2_body,
        out_shape=jax.ShapeDtypeStruct((N_PAD, D_HID), jnp.float32),
    )(agg1p, table1, dinv, b1r, w2p)

    agg2p = _sc_aggregate(table2, srcg, dstg)

    out16 = pl.pallas_call(
        _tc_out_body,
        out_shape=jax.ShapeDtypeStruct((N_PAD, D_HID), jnp.float32),
    )(agg2p, table2, dinv, b2r)

    return out16[:N_NODES, :N_CLASSES]


# speedup vs baseline: 53.9679x; 1.3991x over previous
import functools

import jax
import jax.numpy as jnp
from jax import lax
from jax.experimental import pallas as pl
from jax.experimental.pallas import tpu as pltpu
from jax.experimental.pallas import tpu_sc as plsc

N_NODES = 10000
N_PAD = 10240
N_EDGES = 320000
NW = 32
K = 8
G = 10
NB = K * G
E_PAD = NW * NB * 128
D_FEAT = 128
D_HID = 16
N_CLASSES = 7
ROWS_PER_TILE = N_PAD // 16


def _sc_mesh():
    return plsc.VectorSubcoreMesh(
        core_axis_name="c", subcore_axis_name="s", num_cores=2, num_subcores=16
    )


def _deg_body(dstg, degp, dst_v, ones_v, zbuf_v, deg_sh, sem):
    c = lax.axis_index("c")
    s = lax.axis_index("s")
    wid = c * 16 + s

    def fill_ones(i, carry):
        ones_v[pl.ds(i * 16, 16)] = jnp.ones((16,), jnp.float32)
        return carry

    lax.fori_loop(0, 8, fill_ones, 0)

    def fill_zeros(i, carry):
        zbuf_v[pl.ds(i * 16, 16)] = jnp.zeros((16,), jnp.float32)
        return carry

    lax.fori_loop(0, ROWS_PER_TILE // 16, fill_zeros, 0)

    pltpu.sync_copy(dstg.at[wid], dst_v)
    pltpu.sync_copy(zbuf_v, deg_sh.at[pl.ds(s * ROWS_PER_TILE, ROWS_PER_TILE)])
    plsc.subcore_barrier()

    def body(g, carry):
        descs = [
            pltpu.async_copy(ones_v, deg_sh.at[dst_v.at[g * K + b]], sem, add=True)
            for b in range(K)
        ]
        for d in descs:
            d.wait()
        return carry

    lax.fori_loop(0, G, body, 0)
    plsc.subcore_barrier()

    pltpu.sync_copy(deg_sh.at[pl.ds(s * ROWS_PER_TILE, ROWS_PER_TILE)], zbuf_v)
    pltpu.sync_copy(zbuf_v, degp.at[c, pl.ds(s * ROWS_PER_TILE, ROWS_PER_TILE)])


def _sc_degree(dstg):
    kern = functools.partial(
        pl.kernel,
        out_type=jax.ShapeDtypeStruct((2, N_PAD), jnp.float32),
        mesh=_sc_mesh(),
        scratch_types=[
            pltpu.VMEM((NB, 128), jnp.int32),
            pltpu.VMEM((128,), jnp.float32),
            pltpu.VMEM((ROWS_PER_TILE,), jnp.float32),
            pltpu.VMEM_SHARED((N_PAD,), jnp.float32),
            pltpu.SemaphoreType.DMA,
        ],
        compiler_params=pltpu.CompilerParams(use_tc_tiling_on_sc=False),
    )(_deg_body)
    return kern(dstg)


def _agg_body(table, srcg, dstg, aggp, src_v, dst_v, rows_v, zbuf_v, acc_sh,
              table_sh, gsem, ssem):
    c = lax.axis_index("c")
    s = lax.axis_index("s")
    wid = c * 16 + s

    tile_rows = pl.ds(s * ROWS_PER_TILE, ROWS_PER_TILE)
    pltpu.sync_copy(table.at[tile_rows], zbuf_v)
    pltpu.sync_copy(zbuf_v, table_sh.at[tile_rows])

    def fill_zeros(i, carry):
        zbuf_v[i, :] = jnp.zeros((16,), jnp.float32)
        return carry

    lax.fori_loop(0, ROWS_PER_TILE, fill_zeros, 0)

    pltpu.sync_copy(srcg.at[wid], src_v)
    pltpu.sync_copy(dstg.at[wid], dst_v)
    pltpu.sync_copy(zbuf_v, acc_sh.at[tile_rows])
    plsc.subcore_barrier()

    def issue_gathers(g, setidx):
        for b in range(K):
            pltpu.async_copy(
                table_sh.at[src_v.at[g * K + b]], rows_v.at[setidx, b], gsem
            )

    issue_gathers(0, 0)

    def body(g, carry):
        cur = lax.rem(g, 2)
        nxt = lax.rem(g + 1, 2)

        @pl.when(g + 1 < G)
        def _():
            issue_gathers(g + 1, nxt)

        for b in range(K):
            pltpu.make_async_copy(
                table_sh.at[src_v.at[g * K + b]], rows_v.at[cur, b], gsem
            ).wait()
        descs = [
            pltpu.async_copy(
                rows_v.at[cur, b], acc_sh.at[dst_v.at[g * K + b]], ssem, add=True
            )
            for b in range(K)
        ]
        for d in descs:
            d.wait()
        return carry

    lax.fori_loop(0, G, body, 0)
    plsc.subcore_barrier()

    pltpu.sync_copy(acc_sh.at[tile_rows], zbuf_v)
    pltpu.sync_copy(zbuf_v, aggp.at[c, tile_rows])


def _sc_aggregate(table, srcg, dstg):
    kern = functools.partial(
        pl.kernel,
        out_type=jax.ShapeDtypeStruct((2, N_PAD, D_HID), jnp.float32),
        mesh=_sc_mesh(),
        scratch_types=[
            pltpu.VMEM((NB, 128), jnp.int32),
            pltpu.VMEM((NB, 128), jnp.int32),
            pltpu.VMEM((2, K, 128, D_HID), jnp.float32),
            pltpu.VMEM((ROWS_PER_TILE, D_HID), jnp.float32),
            pltpu.VMEM_SHARED((N_PAD, D_HID), jnp.float32),
            pltpu.VMEM_SHARED((N_PAD, D_HID), jnp.float32),
            pltpu.SemaphoreType.DMA,
            pltpu.SemaphoreType.DMA,
        ],
        compiler_params=pltpu.CompilerParams(use_tc_tiling_on_sc=False),
    )(_agg_body)
    return kern(table, srcg, dstg)


def _tc_dinv_body(degp_ref, dinv_ref):
    deg = degp_ref[0:1, :] + degp_ref[1:2, :] + 1.0
    dinv_ref[...] = lax.rsqrt(deg)


def _tc_table1_body(x_ref, w1_ref, dinv_ref, out_ref):
    h = jnp.dot(x_ref[...], w1_ref[...], preferred_element_type=jnp.float32)
    out_ref[...] = h * dinv_ref[...]


def _tc_table2_body(aggp_ref, t1_ref, dinv_ref, b1_ref, w2_ref, out_ref):
    agg = aggp_ref[0, :, :] + aggp_ref[1, :, :] + t1_ref[...]
    z1 = jnp.maximum(agg * dinv_ref[...] + b1_ref[...], 0.0)
    h2 = jnp.dot(z1, w2_ref[...], preferred_element_type=jnp.float32)
    out_ref[...] = h2 * dinv_ref[...]


def _tc_out_body(aggp_ref, t2_ref, dinv_ref, b2_ref, out_ref):
    agg = aggp_ref[0, :, :] + aggp_ref[1, :, :] + t2_ref[...]
    z = agg * dinv_ref[...] + b2_ref[...]
    col = lax.broadcasted_iota(jnp.int32, (N_PAD, D_HID), 1)
    zm = jnp.where(col < N_CLASSES, z, -1e30)
    m = jnp.max(zm, axis=1, keepdims=True)
    ssum = jnp.sum(jnp.exp(zm - m), axis=1, keepdims=True)
    out_ref[...] = z - m - jnp.log(ssum)


def kernel(x, edge_index, W1, b1, W2, b2):
    src = edge_index[0]
    dst = edge_index[1]
    pad = jnp.full((E_PAD - N_EDGES,), N_PAD - 1, dtype=jnp.int32)
    srcg = jnp.concatenate([src, pad]).reshape(NW, NB, 128)
    dstg = jnp.concatenate([dst, pad]).reshape(NW, NB, 128)
    x_pad = jnp.pad(x, ((0, N_PAD - N_NODES), (0, 0)))
    w2p = jnp.pad(W2, ((0, 0), (0, D_HID - N_CLASSES)))
    b1r = b1.reshape(1, D_HID)
    b2r = jnp.pad(b2, (0, D_HID - N_CLASSES)).reshape(1, D_HID)

    degp = _sc_degree(dstg)

    dinv_row = pl.pallas_call(
        _tc_dinv_body,
        out_shape=jax.ShapeDtypeStruct((1, N_PAD), jnp.float32),
    )(degp)
    dinv = dinv_row.reshape(N_PAD, 1)

    table1 = pl.pallas_call(
        _tc_table1_body,
        out_shape=jax.ShapeDtypeStruct((N_PAD, D_HID), jnp.float32),
    )(x_pad, W1, dinv)

    agg1p = _sc_aggregate(table1, srcg, dstg)

    table2 = pl.pallas_call(
        _tc_table2_body,
        out_shape=jax.ShapeDtypeStruct((N_PAD, D_HID), jnp.float32),
    )(agg1p, table1, dinv, b1r, w2p)

    agg2p = _sc_aggregate(table2, srcg, dstg)

    out16 = pl.pallas_call(
        _tc_out_body,
        out_shape=jax.ShapeDtypeStruct((N_PAD, D_HID), jnp.float32),
    )(agg2p, table2, dinv, b2r)

    return out16[:N_NODES, :N_CLASSES]
